# Initial kernel scaffold; baseline (speedup 1.0000x reference)
import functools

import jax
import jax.numpy as jnp
from jax import lax
from jax.experimental import pallas as pl
from jax.experimental.pallas import tpu as pltpu
from jax.experimental.pallas import tpu_sc as plsc

_NC = 2
_NS = 16
_L = 16
_NW = _NC * _NS

_BE = 80


def _node_logits(feat, w):
  n, hd = feat.shape
  h2 = w.shape[1]

  def body(feat_ref, w_ref, eler_ref, svec_ref):
    eler = jnp.dot(feat_ref[...], w_ref[...],
                   preferred_element_type=jnp.float32)
    eler_ref[...] = eler
    cm = jnp.max(eler, axis=0, keepdims=True)
    h = h2 // 2
    ssum = cm[:, :h] + cm[:, h:]
    s = jnp.where(ssum > 0, ssum, 0.2 * ssum)
    svec_ref[...] = jnp.concatenate([s, jnp.zeros_like(s)], axis=1)

  return pl.pallas_call(
      body,
      out_shape=(
          jax.ShapeDtypeStruct((n, h2), jnp.float32),
          jax.ShapeDtypeStruct((1, h2), jnp.float32),
      ),
  )(feat, w)


@functools.lru_cache(maxsize=None)
def _edge_call(n, e, h, hd):
  ept = e // _NW
  nb = ept // _BE
  assert ept % _BE == 0 and e % _NW == 0
  rpt = n // _NS
  assert n % _NS == 0
  h2 = 2 * h
  groups = _BE // _L
  nvr = hd // _L
  vper = hd // (h * _L)

  mesh = plsc.VectorSubcoreMesh(core_axis_name="c", subcore_axis_name="s")

  @functools.partial(
      pl.kernel,
      out_type=(
          jax.ShapeDtypeStruct((_NC, n, hd), jnp.float32),
          jax.ShapeDtypeStruct((_NC, n, h), jnp.float32),
      ),
      mesh=mesh,
      scratch_types=[
          pltpu.VMEM((n, h2), jnp.float32),
          pltpu.VMEM((nb, _BE), jnp.int32),
          pltpu.VMEM((nb, _BE), jnp.int32),
          pltpu.VMEM((_BE, hd), jnp.float32),
          pltpu.VMEM((_BE, h), jnp.float32),
          pltpu.VMEM((1, h2), jnp.float32),
          pltpu.VMEM((25, hd), jnp.float32),
          pltpu.VMEM_SHARED((n, hd), jnp.float32),
          pltpu.VMEM_SHARED((n, h), jnp.float32),
          pltpu.SemaphoreType.DMA,
      ],
  )
  def edge_kernel(eler_hbm, src_hbm, dst_hbm, feat_hbm, svec_hbm,
                  acc_out, den_out,
                  eler_v, src_v, dst_v, rows_v, expv_v, svec_v, zbuf_v,
                  acc_sh, den_sh, sem):
    c = lax.axis_index("c")
    s = lax.axis_index("s")
    wid = c * _NS + s

    pltpu.sync_copy(eler_hbm, eler_v)
    pltpu.sync_copy(svec_hbm, svec_v)
    pltpu.sync_copy(src_hbm.at[pl.ds(wid * nb, nb)], src_v)
    pltpu.sync_copy(dst_hbm.at[pl.ds(wid * nb, nb)], dst_v)

    zv = jnp.zeros((_L,), jnp.float32)
    for i in range(25):
      for j in range(nvr):
        zbuf_v[i, pl.ds(j * _L, _L)] = zv
    for i in range(rpt // 25):
      pltpu.sync_copy(zbuf_v, acc_sh.at[pl.ds(s * rpt + i * 25, 25)])
      pltpu.sync_copy(zbuf_v.at[:, pl.ds(0, h)],
                      den_sh.at[pl.ds(s * rpt + i * 25, 25)])
    plsc.subcore_barrier()

    def batch_body(b, carry):
      pltpu.async_copy(feat_hbm.at[src_v.at[b]], rows_v, sem).wait()

      for g in range(groups):
        base = g * _L
        src16 = src_v[b, pl.ds(base, _L)]
        dst16 = dst_v[b, pl.ds(base, _L)]
        rowidx = lax.iota(jnp.int32, _L) + base
        for hh in range(h):
          col_l = jnp.full((_L,), hh, jnp.int32)
          col_r = jnp.full((_L,), h + hh, jnp.int32)
          el = plsc.load_gather(eler_v, [src16, col_l])
          er = plsc.load_gather(eler_v, [dst16, col_r])
          lo = el + er
          lo = jnp.where(lo > 0, lo, 0.2 * lo)
          expe = jnp.exp(lo - svec_v[0, hh])
          plsc.store_scatter(expv_v, [rowidx, col_l], expe)

      for ei in range(_BE):
        for hh in range(h):
          av = jnp.full((_L,), expv_v[ei, hh], jnp.float32)
          for k in range(vper):
            j = (hh * vper + k) * _L
            rows_v[ei, pl.ds(j, _L)] = rows_v[ei, pl.ds(j, _L)] * av

      pltpu.sync_copy(rows_v, acc_sh.at[dst_v.at[b]], add=True)
      pltpu.sync_copy(expv_v, den_sh.at[dst_v.at[b]], add=True)
      return carry

    lax.fori_loop(0, nb, batch_body, 0)

    plsc.subcore_barrier()
    ob = s * rpt
    pltpu.sync_copy(acc_sh.at[pl.ds(ob, rpt)], acc_out.at[c, pl.ds(ob, rpt)])
    pltpu.sync_copy(den_sh.at[pl.ds(ob, rpt)], den_out.at[c, pl.ds(ob, rpt)])

  return edge_kernel


def _finalize(acc, den, p, bias2d):
  _, n, hd = acc.shape
  h = den.shape[2]
  bk = 400

  def body(acc_ref, den_ref, p_ref, b_ref, out_ref):
    a = acc_ref[0] + acc_ref[1]
    d = den_ref[0] + den_ref[1]
    dx = jnp.dot(d, p_ref[...], preferred_element_type=jnp.float32)
    dx = jnp.where(dx == 0.0, 1.0, dx)
    out_ref[...] = a / dx + b_ref[...]

  return pl.pallas_call(
      body,
      grid=(n // bk,),
      in_specs=[
          pl.BlockSpec((2, bk, hd), lambda i: (0, i, 0)),
          pl.BlockSpec((2, bk, h), lambda i: (0, i, 0)),
          pl.BlockSpec((h, hd), lambda i: (0, 0)),
          pl.BlockSpec((1, hd), lambda i: (0, 0)),
      ],
      out_specs=pl.BlockSpec((bk, hd), lambda i: (i, 0)),
      out_shape=jax.ShapeDtypeStruct((n, hd), jnp.float32),
  )(acc, den, p, bias2d)


def kernel(feat, edge_index, attn_l, attn_r, bias):
  n, hd = feat.shape
  h, d = attn_l.shape[1], attn_l.shape[2]
  e = edge_index.shape[1]

  head_of = (jnp.arange(hd)[:, None] // d == jnp.arange(h)[None, :])
  head_of = head_of.astype(jnp.float32)
  wl = head_of * attn_l.reshape(hd)[:, None]
  wr = head_of * attn_r.reshape(hd)[:, None]
  w = jnp.concatenate([wl, wr], axis=1)

  eler, svec = _node_logits(feat, w)

  nb = e // (_NW * _BE)
  src2d = edge_index[0].reshape(_NW * nb, _BE)
  dst2d = edge_index[1].reshape(_NW * nb, _BE)

  acc, den = _edge_call(n, e, h, hd)(eler, src2d, dst2d, feat, svec)

  out = _finalize(acc, den, head_of.T, bias.reshape(1, hd))
  return out.reshape(n, h, d)


# speedup vs baseline: 48.8208x; 48.8208x over previous
import functools

import jax
import jax.numpy as jnp
from jax import lax
from jax.experimental import pallas as pl
from jax.experimental.pallas import tpu as pltpu
from jax.experimental.pallas import tpu_sc as plsc

_NC = 2
_NS = 16
_L = 16

_BE = 80
_NBP = 256


def _node_logits(feat, w):
  n, hd = feat.shape
  h2 = w.shape[1]
  h = h2 // 2

  def body(feat_ref, w_ref, eler_ref, svec_ref):
    y = jnp.dot(feat_ref[...], w_ref[...],
                preferred_element_type=jnp.float32)
    eler_ref[...] = jnp.concatenate(
        [y, jnp.zeros((n, 16 - h2), jnp.float32)], axis=1)
    cm = jnp.max(y, axis=0, keepdims=True)
    ssum = cm[:, :h] + cm[:, h:]
    s = jnp.where(ssum > 0, ssum, 0.2 * ssum)
    svec_ref[...] = jnp.concatenate(
        [s, jnp.zeros((1, 16 - h), jnp.float32)], axis=1)

  return pl.pallas_call(
      body,
      out_shape=(
          jax.ShapeDtypeStruct((n, 16), jnp.float32),
          jax.ShapeDtypeStruct((1, 16), jnp.float32),
      ),
  )(feat, w)


@functools.lru_cache(maxsize=None)
def _edge_call(n, e, h, hd):
  hh2 = h // _NC
  hw = hd // _NC
  h2 = 2 * h
  ept = e // _NS
  nb = ept // _BE
  assert ept % _BE == 0 and nb <= _NBP
  npad = ((n + _NS * 16 - 1) // (_NS * 16)) * (_NS * 16)
  rpt = npad // _NS
  groups = _BE // _L
  nvr = hw // _L
  vper = hw // (hh2 * _L)
  mesh = plsc.VectorSubcoreMesh(core_axis_name="c", subcore_axis_name="s")

  @functools.partial(
      pl.kernel,
      out_type=(
          jax.ShapeDtypeStruct((_NC, npad, hw), jnp.float32),
          jax.ShapeDtypeStruct((_NC, npad, 16), jnp.float32),
      ),
      mesh=mesh,
      scratch_types=[
          pltpu.VMEM((_NBP, _BE), jnp.int32),
          pltpu.VMEM((_NBP, _BE), jnp.int32),
          pltpu.VMEM((_BE, 16), jnp.float32),
          pltpu.VMEM((_BE, 16), jnp.float32),
          pltpu.VMEM((_BE, hw), jnp.float32),
          pltpu.VMEM((_BE, 16), jnp.float32),
          pltpu.VMEM((1, 16), jnp.float32),
          pltpu.VMEM((16, hw), jnp.float32),
          pltpu.VMEM((16, 16), jnp.float32),
          pltpu.VMEM_SHARED((npad, hw), jnp.float32),
          pltpu.VMEM_SHARED((npad, 16), jnp.float32),
          pltpu.SemaphoreType.DMA,
      ],
      compiler_params=pltpu.CompilerParams(
          needs_layout_passes=False, use_tc_tiling_on_sc=False),
  )
  def edge_kernel(eler2_hbm, srcoff_hbm, dst_hbm, feat2_hbm, svec_hbm,
                  acc_out, den_out,
                  srcoff_v, dst_v, elb_v, erb_v, rows_v, expv_v, svec_v,
                  zbuf_v, zbufd_v, acc_sh, den_sh, sem):
    c = lax.axis_index("c")
    s = lax.axis_index("s")

    pltpu.sync_copy(svec_hbm, svec_v)
    pltpu.sync_copy(srcoff_hbm.at[pl.ds((c * _NS + s) * _NBP, _NBP)],
                    srcoff_v)
    pltpu.sync_copy(dst_hbm.at[pl.ds(s * _NBP, _NBP)], dst_v)

    zv = jnp.zeros((_L,), jnp.float32)
    for i in range(16):
      for j in range(nvr):
        zbuf_v[i, pl.ds(j * _L, _L)] = zv
      zbufd_v[i, pl.ds(0, _L)] = zv
    for ei in range(_BE):
      expv_v[ei, pl.ds(0, _L)] = zv
    for i in range(rpt // 16):
      pltpu.sync_copy(zbuf_v, acc_sh.at[pl.ds(s * rpt + i * 16, 16)])
      pltpu.sync_copy(zbufd_v, den_sh.at[pl.ds(s * rpt + i * 16, 16)])
    plsc.subcore_barrier()

    svec16 = svec_v[0, pl.ds(0, _L)]
    lane = lax.iota(jnp.int32, _L)
    zero16 = jnp.zeros((_L,), jnp.float32)
    shifts = [
        jnp.full((_L,), jnp.sum(jnp.where(lane == c * hh2 + j, svec16,
                                          zero16)), jnp.float32)
        for j in range(hh2)
    ]
    rowiota = lane

    def batch_body(b, carry):
      fcp = pltpu.async_copy(feat2_hbm.at[srcoff_v.at[b]], rows_v, sem)
      ecp = pltpu.async_copy(eler2_hbm.at[srcoff_v.at[b]], elb_v, sem)
      rcp = pltpu.async_copy(eler2_hbm.at[dst_v.at[b]], erb_v, sem)
      fcp.wait()
      ecp.wait()
      rcp.wait()
      for g in range(groups):
        base = g * _L
        rowidx = rowiota + base
        expes = []
        for j in range(hh2):
          col_l = jnp.full((_L,), c * hh2 + j, jnp.int32)
          col_r = col_l + h
          el = plsc.load_gather(elb_v, [rowidx, col_l])
          er = plsc.load_gather(erb_v, [rowidx, col_r])
          lo = el + er
          lo = jnp.where(lo > 0, lo, 0.2 * lo)
          expe = jnp.exp(lo - shifts[j])
          plsc.store_scatter(expv_v, [rowidx, jnp.full((_L,), j, jnp.int32)],
                             expe)
          expes.append(expe)
        for i in range(_L):
          ei = base + i
          for j in range(hh2):
            av = jnp.full((_L,), expes[j][i], jnp.float32)
            for k in range(vper):
              col = (j * vper + k) * _L
              rows_v[ei, pl.ds(col, _L)] = rows_v[ei, pl.ds(col, _L)] * av
      pltpu.sync_copy(rows_v, acc_sh.at[dst_v.at[b]], add=True)
      pltpu.sync_copy(expv_v, den_sh.at[dst_v.at[b]], add=True)
      return carry

    lax.fori_loop(0, nb, batch_body, 0)

    plsc.subcore_barrier()
    ob = s * rpt
    pltpu.sync_copy(acc_sh.at[pl.ds(ob, rpt)], acc_out.at[c, pl.ds(ob, rpt)])
    pltpu.sync_copy(den_sh.at[pl.ds(ob, rpt)], den_out.at[c, pl.ds(ob, rpt)])

  return edge_kernel


def _finalize(acc, den, p0, p1, bias2d, n):
  hw = acc.shape[2]
  pw = den.shape[2]
  bk = 400

  def body(acc_ref, den_ref, p0_ref, p1_ref, b_ref, out_ref):
    num = jnp.concatenate([acc_ref[0], acc_ref[1]], axis=1)
    dx = (jnp.dot(den_ref[0], p0_ref[...], preferred_element_type=jnp.float32)
          + jnp.dot(den_ref[1], p1_ref[...],
                    preferred_element_type=jnp.float32))
    dx = jnp.where(dx == 0.0, 1.0, dx)
    out_ref[...] = num / dx + b_ref[...]

  return pl.pallas_call(
      body,
      grid=(n // bk,),
      in_specs=[
          pl.BlockSpec((2, bk, hw), lambda i: (0, i, 0)),
          pl.BlockSpec((2, bk, pw), lambda i: (0, i, 0)),
          pl.BlockSpec((pw, 2 * hw), lambda i: (0, 0)),
          pl.BlockSpec((pw, 2 * hw), lambda i: (0, 0)),
          pl.BlockSpec((1, 2 * hw), lambda i: (0, 0)),
      ],
      out_specs=pl.BlockSpec((bk, 2 * hw), lambda i: (i, 0)),
      out_shape=jax.ShapeDtypeStruct((n, 2 * hw), jnp.float32),
  )(acc, den, p0, p1, bias2d)


def kernel(feat, edge_index, attn_l, attn_r, bias):
  n, hd = feat.shape
  h, d = attn_l.shape[1], attn_l.shape[2]
  e = edge_index.shape[1]
  hh2 = h // _NC

  head_of = (jnp.arange(hd)[:, None] // d == jnp.arange(h)[None, :])
  head_of = head_of.astype(jnp.float32)
  wl = head_of * attn_l.reshape(hd)[:, None]
  wr = head_of * attn_r.reshape(hd)[:, None]
  w = jnp.concatenate([wl, wr], axis=1)

  eler, svec = _node_logits(feat, w)
  eler2 = jnp.concatenate([eler, eler], axis=0)

  feat2 = feat.reshape(n, _NC, hd // _NC).transpose(1, 0, 2)
  feat2 = feat2.reshape(_NC * n, hd // _NC)

  nb = (e // _NS) // _BE
  src3 = edge_index[0].reshape(_NS, nb, _BE)
  dst3 = edge_index[1].reshape(_NS, nb, _BE)
  pad = ((0, 0), (0, _NBP - nb), (0, 0))
  srcp = jnp.pad(src3, pad).reshape(_NS * _NBP, _BE)
  dstp = jnp.pad(dst3, pad).reshape(_NS * _NBP, _BE)
  srcoff = jnp.concatenate([srcp, srcp + n], axis=0)

  acc, den = _edge_call(n, e, h, hd)(eler2, srcoff, dstp, feat2, svec)

  lanes = jnp.arange(hd)[None, :] // d
  js = jnp.arange(16)[:, None]
  p0 = ((lanes == js) & (js < hh2)).astype(jnp.float32)
  p1 = ((lanes == js + hh2) & (js < hh2)).astype(jnp.float32)
  out = _finalize(acc, den, p0, p1, bias.reshape(1, hd), n)
  return out.reshape(n, h, d)
